# baseline (speedup 1.0000x reference)
import functools

import jax
import jax.numpy as jnp
from jax.experimental import pallas as pl
from jax.experimental.pallas import tpu as pltpu

MARGIN = 10.0


def _loss_kernel(labels_ref, emb_ref, cmat_ref, ones_ref, bb_ref, out_ref,
                 m1_ref, m2_ref, w_ref, *, n_blocks, k):
    i = pl.program_id(0)

    @pl.when(i == 0)
    def _init():
        m1_ref[...] = jnp.zeros_like(m1_ref)
        m2_ref[...] = jnp.zeros_like(m2_ref)
        w_ref[...] = jnp.zeros_like(w_ref)

    e = emb_ref[...]
    lab = labels_ref[0]
    bb = bb_ref[0:1, :]

    p = jax.lax.dot_general(
        e, cmat_ref[...], (((1,), (0,)), ((), ())),
        preferred_element_type=jnp.float32,
    ) + jax.lax.dot_general(
        e * e, ones_ref[...], (((1,), (0,)), ((), ())),
        preferred_element_type=jnp.float32,
    )

    d2 = jnp.maximum(p + bb, 1e-12)
    d = d2 * jax.lax.rsqrt(d2)

    oht = (lab == jax.lax.broadcasted_iota(jnp.int32, (k, 1), 0)
           ).astype(jnp.float32)

    dn = (((1,), (0,)), ((), ()))
    m1_ref[...] += jax.lax.dot_general(
        oht, d, dn, preferred_element_type=jnp.float32)
    m2_ref[...] += jax.lax.dot_general(
        oht, d2, dn, preferred_element_type=jnp.float32)
    w_ref[...] += jnp.sum(oht, axis=1, keepdims=True)

    @pl.when(i == n_blocks - 1)
    def _finish():
        eye = (jax.lax.broadcasted_iota(jnp.int32, (k, k), 0)
               == jax.lax.broadcasted_iota(jnp.int32, (k, k), 1)
               ).astype(jnp.float32)
        m1 = m1_ref[...]
        m2 = m2_ref[...]
        counts = w_ref[:, 0]
        od = jnp.sum(m1 * eye, axis=1)
        sd = jnp.sum(m1, axis=1)
        a_sum = jnp.sum(m2 * eye, axis=1)
        ssum_d2 = jnp.sum(m2, axis=1)
        s_tot = (100.0 * k) * counts - 20.0 * sd + ssum_d2
        rep_diag = 100.0 * counts - 20.0 * od + a_sum
        attr = a_sum / jnp.maximum(counts, 1.0)
        rep = (s_tot - rep_diag) / jnp.maximum(counts * (k - 1), 1.0)
        valid = counts > 0.0
        n_valid = jnp.sum(valid.astype(jnp.float32))
        total = (jnp.sum(jnp.where(valid, attr, 0.0))
                 + jnp.sum(jnp.where(valid, rep, 0.0))) / n_valid
        out_ref[...] = total[None, None]


def kernel(embeddings, cluster_labels, centroids):
    n, d_feat = embeddings.shape
    k = centroids.shape[0]
    block = 10000
    n_blocks = n // block
    assert n_blocks * block == n

    labels3 = jnp.asarray(cluster_labels, jnp.int32).reshape(n_blocks, 1, block)
    emb_bf = embeddings.astype(jnp.bfloat16)
    cmat = (-2.0 * centroids.T).astype(jnp.bfloat16)
    onesmat = jnp.ones((d_feat, k), jnp.bfloat16)
    bbrow = jnp.sum(centroids * centroids, axis=1)[None, :]

    out = pl.pallas_call(
        functools.partial(_loss_kernel, n_blocks=n_blocks, k=k),
        grid=(n_blocks,),
        in_specs=[
            pl.BlockSpec((1, 1, block), lambda i: (i, 0, 0)),
            pl.BlockSpec((block, d_feat), lambda i: (i, 0)),
            pl.BlockSpec((d_feat, k), lambda i: (0, 0)),
            pl.BlockSpec((d_feat, k), lambda i: (0, 0)),
            pl.BlockSpec((1, k), lambda i: (0, 0)),
        ],
        out_specs=pl.BlockSpec((1, 1), lambda i: (0, 0)),
        out_shape=jax.ShapeDtypeStruct((1, 1), jnp.float32),
        scratch_shapes=[
            pltpu.VMEM((k, k), jnp.float32),
            pltpu.VMEM((k, k), jnp.float32),
            pltpu.VMEM((k, 1), jnp.float32),
        ],
    )(labels3, emb_bf, cmat, onesmat, bbrow)
    return out[0, 0]
